# baseline (speedup 1.0000x reference)
import functools

import jax
import jax.numpy as jnp
from jax import lax
from jax.experimental import pallas as pl
from jax.experimental.pallas import tpu as pltpu
from jax.experimental.pallas import tpu_sc as plsc

L = 200
LP = 208
NIDX = 3 * LP + 16
NFLAT = 2 * NIDX
ICHUNK = 128
NDMA = NFLAT // ICHUNK
NVREG = LP // 16
NUMD = 1000000
_BASE = (0, LP, 2 * LP)
_DOC_OFF = 3 * LP


def _sqrt16(x):
    xi = lax.bitcast_convert_type(x, jnp.int32)
    yi = jnp.int32(0x5F3759DF) - lax.shift_right_logical(xi, 1)
    y = lax.bitcast_convert_type(yi, jnp.float32)
    for _ in range(3):
        y = y * (1.5 - 0.5 * x * y * y)
    return jnp.where(x > 0.0, x * y, 0.0)


def _body(nb_hbm, nd_hbm, gl_hbm, gd_hbm, rel_hbm, doc_hbm, pos_hbm, out_hbm,
          nb_v, gl_v, rel_v, doc_v, nd_v, gd_v,
          flat_v, rows_v, out_v, sem, sem2):
    cid = lax.axis_index("c")
    sid = lax.axis_index("s")

    @pl.when(jnp.logical_and(cid == 0, sid == 0))
    def _():
        cp_ids = [
            pltpu.async_copy(nb_hbm, nb_v.at[pl.ds(0, L)], sem),
            pltpu.async_copy(gl_hbm, gl_v.at[pl.ds(0, L)], sem),
            pltpu.async_copy(rel_hbm, rel_v.at[pl.ds(0, L)], sem),
            pltpu.async_copy(doc_hbm, doc_v, sem),
        ]
        cp_nd = pltpu.async_copy(nd_hbm, nd_v.at[pl.ds(0, L)], sem2)
        cp_gd = pltpu.async_copy(gd_hbm, gd_v.at[pl.ds(0, L)], sem2)
        for cp in cp_ids:
            cp.wait()

        lane = lax.iota(jnp.int32, 16)
        tail_mask = lane < (L - 16 * (NVREG - 1))

        def put(f, v):
            flat_v[f // ICHUNK, pl.ds(f % ICHUNK, 16)] = v
            f += NIDX
            flat_v[f // ICHUNK, pl.ds(f % ICHUNK, 16)] = v + NUMD

        for base, buf in zip(_BASE, (nb_v, gl_v, rel_v)):
            for c in range(NVREG):
                v = buf[pl.ds(16 * c, 16)]
                if c == NVREG - 1:
                    v = jnp.where(tail_mask, v, 0)
                put(base + 16 * c, v)
        put(_DOC_OFF, doc_v[...])

        copies = [
            pltpu.async_copy(pos_hbm.at[flat_v.at[j]],
                             rows_v.at[pl.ds(ICHUNK * j, ICHUNK)], sem)
            for j in range(NDMA)
        ]
        cp_nd.wait()
        cp_gd.wait()
        for cp in copies:
            cp.wait()

        dnums = lax.GatherDimensionNumbers(
            offset_dims=(), collapsed_slice_dims=(0,), start_index_map=(0,))

        def hsum16(v):
            for sh in (8, 4, 2, 1):
                perm = (lane + sh) & 15
                v = v + lax.gather(
                    v, perm[:, None], dnums, slice_sizes=(1,),
                    mode=lax.GatherScatterMode.PROMISE_IN_BOUNDS)
            return v

        doc_x = rows_v[pl.ds(_DOC_OFF, 16)]
        doc_y = rows_v[pl.ds(NIDX + _DOC_OFF, 16)]

        def chunk_xy(base, c):
            xv = rows_v[pl.ds(base + 16 * c, 16)]
            yv = rows_v[pl.ds(NIDX + base + 16 * c, 16)]
            return xv, yv

        acc = jnp.zeros((16,), jnp.float32)
        for base, dist_v in ((_BASE[0], nd_v), (_BASE[1], gd_v)):
            for c in range(NVREG):
                xv, yv = chunk_xy(base, c)
                dx = xv - doc_x
                dy = yv - doc_y
                d2 = dx * dx + dy * dy
                t = d2 - dist_v[pl.ds(16 * c, 16)]
                t2 = t * t
                if c == NVREG - 1:
                    t2 = jnp.where(tail_mask, t2, 0.0)
                acc = acc + t2

        sx = jnp.zeros((16,), jnp.float32)
        sy = jnp.zeros((16,), jnp.float32)
        for c in range(NVREG):
            xv, yv = chunk_xy(_BASE[2], c)
            if c == NVREG - 1:
                xv = jnp.where(tail_mask, xv, 0.0)
                yv = jnp.where(tail_mask, yv, 0.0)
            sx = sx + xv
            sy = sy + yv
        cxv = hsum16(sx) * (1.0 / L)
        cyv = hsum16(sy) * (1.0 / L)

        for c in range(NVREG):
            xv, yv = chunk_xy(_BASE[2], c)
            dx = xv - cxv
            dy = yv - cyv
            d = _sqrt16(dx * dx + dy * dy)
            if c == NVREG - 1:
                d = jnp.where(tail_mask, d, 0.0)
            acc = acc + d

        out_v[...] = hsum16(acc)
        pltpu.sync_copy(out_v, out_hbm)


_sc_call = functools.partial(
    pl.kernel,
    out_type=jax.ShapeDtypeStruct((16,), jnp.float32),
    mesh=plsc.VectorSubcoreMesh(core_axis_name="c", subcore_axis_name="s",
                                num_cores=1),
    scratch_types=[
        pltpu.VMEM((LP,), jnp.int32),
        pltpu.VMEM((LP,), jnp.int32),
        pltpu.VMEM((LP,), jnp.int32),
        pltpu.VMEM((16,), jnp.int32),
        pltpu.VMEM((LP,), jnp.float32),
        pltpu.VMEM((LP,), jnp.float32),
        pltpu.VMEM((NDMA, ICHUNK), jnp.int32),
        pltpu.VMEM((NFLAT,), jnp.float32),
        pltpu.VMEM((16,), jnp.float32),
        pltpu.SemaphoreType.DMA,
        pltpu.SemaphoreType.DMA,
    ],
)(_body)


def kernel(doc_i, neighbours, neighbour_distances, global_docs,
           global_distances, related_docs, positions):
    doc16 = jnp.full((16,), doc_i, jnp.int32)
    pos_flat = jnp.concatenate([positions[:, 0], positions[:, 1]])
    out = _sc_call(neighbours.astype(jnp.int32), neighbour_distances,
                   global_docs.astype(jnp.int32), global_distances,
                   related_docs.astype(jnp.int32), doc16, pos_flat)
    return out[0]


# speedup vs baseline: 2.3602x; 2.3602x over previous
import functools

import jax
import jax.numpy as jnp
from jax import lax
from jax.experimental import pallas as pl
from jax.experimental.pallas import tpu as pltpu
from jax.experimental.pallas import tpu_sc as plsc

L = 200
LP = 208
NIDX = 3 * LP + 16
NFLAT = 2 * NIDX
ICHUNK = 128
NDMA = NFLAT // ICHUNK
NVREG = LP // 16
NUMD = 1000000
_BASE = (0, LP, 2 * LP)
_DOC_OFF = 3 * LP


def _sqrt16(x):
    xi = lax.bitcast_convert_type(x, jnp.int32)
    yi = jnp.int32(0x5F3759DF) - lax.shift_right_logical(xi, 1)
    y = lax.bitcast_convert_type(yi, jnp.float32)
    for _ in range(3):
        y = y * (1.5 - 0.5 * x * y * y)
    return jnp.where(x > 0.0, x * y, 0.0)


def _body(nb_hbm, nd_hbm, gl_hbm, gd_hbm, rel_hbm, doc_hbm, pos_hbm, out_hbm,
          nb_v, gl_v, rel_v, doc_v, nd_v, gd_v,
          flat_v, rows_v, out_v, sem, sem2):
    cid = lax.axis_index("c")
    sid = lax.axis_index("s")

    @pl.when(jnp.logical_and(cid == 0, sid == 0))
    def _():
        cp_ids = [
            pltpu.async_copy(nb_hbm, nb_v.at[pl.ds(0, L)], sem),
            pltpu.async_copy(gl_hbm, gl_v.at[pl.ds(0, L)], sem),
            pltpu.async_copy(rel_hbm, rel_v.at[pl.ds(0, L)], sem),
            pltpu.async_copy(doc_hbm, doc_v, sem),
        ]
        cp_nd = pltpu.async_copy(nd_hbm, nd_v.at[pl.ds(0, L)], sem2)
        cp_gd = pltpu.async_copy(gd_hbm, gd_v.at[pl.ds(0, L)], sem2)
        for cp in cp_ids:
            cp.wait()

        lane = lax.iota(jnp.int32, 16)
        tail_mask = lane < (L - 16 * (NVREG - 1))

        def put(f, v):
            flat_v[f // ICHUNK, pl.ds(f % ICHUNK, 16)] = v
            f += NIDX
            flat_v[f // ICHUNK, pl.ds(f % ICHUNK, 16)] = v + NUMD

        for base, buf in zip(_BASE, (nb_v, gl_v, rel_v)):
            for c in range(NVREG):
                v = buf[pl.ds(16 * c, 16)]
                if c == NVREG - 1:
                    v = jnp.where(tail_mask, v, 0)
                put(base + 16 * c, v)
        put(_DOC_OFF, doc_v[...])

        copies = [
            pltpu.async_copy(pos_hbm.at[flat_v.at[j]],
                             rows_v.at[pl.ds(ICHUNK * j, ICHUNK)], sem)
            for j in range(NDMA)
        ]
        cp_nd.wait()
        cp_gd.wait()
        for cp in copies:
            cp.wait()

        dnums = lax.GatherDimensionNumbers(
            offset_dims=(), collapsed_slice_dims=(0,), start_index_map=(0,))

        def hsum16(v):
            for sh in (8, 4, 2, 1):
                perm = (lane + sh) & 15
                v = v + lax.gather(
                    v, perm[:, None], dnums, slice_sizes=(1,),
                    mode=lax.GatherScatterMode.PROMISE_IN_BOUNDS)
            return v

        doc_x = rows_v[pl.ds(_DOC_OFF, 16)]
        doc_y = rows_v[pl.ds(NIDX + _DOC_OFF, 16)]

        def chunk_xy(base, c):
            xv = rows_v[pl.ds(base + 16 * c, 16)]
            yv = rows_v[pl.ds(NIDX + base + 16 * c, 16)]
            return xv, yv

        acc = jnp.zeros((16,), jnp.float32)
        for base, dist_v in ((_BASE[0], nd_v), (_BASE[1], gd_v)):
            for c in range(NVREG):
                xv, yv = chunk_xy(base, c)
                dx = xv - doc_x
                dy = yv - doc_y
                d2 = dx * dx + dy * dy
                t = d2 - dist_v[pl.ds(16 * c, 16)]
                t2 = t * t
                if c == NVREG - 1:
                    t2 = jnp.where(tail_mask, t2, 0.0)
                acc = acc + t2

        sx = jnp.zeros((16,), jnp.float32)
        sy = jnp.zeros((16,), jnp.float32)
        for c in range(NVREG):
            xv, yv = chunk_xy(_BASE[2], c)
            if c == NVREG - 1:
                xv = jnp.where(tail_mask, xv, 0.0)
                yv = jnp.where(tail_mask, yv, 0.0)
            sx = sx + xv
            sy = sy + yv
        cxv = hsum16(sx) * (1.0 / L)
        cyv = hsum16(sy) * (1.0 / L)

        for c in range(NVREG):
            xv, yv = chunk_xy(_BASE[2], c)
            dx = xv - cxv
            dy = yv - cyv
            d = _sqrt16(dx * dx + dy * dy)
            if c == NVREG - 1:
                d = jnp.where(tail_mask, d, 0.0)
            acc = acc + d

        out_v[...] = hsum16(acc)
        pltpu.sync_copy(out_v, out_hbm)


_sc_call = functools.partial(
    pl.kernel,
    out_type=jax.ShapeDtypeStruct((16,), jnp.float32),
    mesh=plsc.VectorSubcoreMesh(core_axis_name="c", subcore_axis_name="s",
                                num_cores=1),
    scratch_types=[
        pltpu.VMEM((LP,), jnp.int32),
        pltpu.VMEM((LP,), jnp.int32),
        pltpu.VMEM((LP,), jnp.int32),
        pltpu.VMEM((16,), jnp.int32),
        pltpu.VMEM((LP,), jnp.float32),
        pltpu.VMEM((LP,), jnp.float32),
        pltpu.VMEM((NDMA, ICHUNK), jnp.int32),
        pltpu.VMEM((NFLAT,), jnp.float32),
        pltpu.VMEM((16,), jnp.float32),
        pltpu.SemaphoreType.DMA,
        pltpu.SemaphoreType.DMA,
    ],
)(_body)


def kernel(doc_i, neighbours, neighbour_distances, global_docs,
           global_distances, related_docs, positions):
    doc16 = jnp.full((16,), doc_i, jnp.int32)
    pos_flat = positions.T.reshape(-1)
    out = _sc_call(neighbours.astype(jnp.int32), neighbour_distances,
                   global_docs.astype(jnp.int32), global_distances,
                   related_docs.astype(jnp.int32), doc16, pos_flat)
    return out[0]
